# Initial kernel scaffold; baseline (speedup 1.0000x reference)
import jax
import jax.numpy as jnp
from jax.experimental import pallas as pl


def kernel(mailbox_h, mailbox_c, U_f_w, U_f_b, U_iou_w, b_iou):
    raise NotImplementedError("write your pallas kernel here")


# speedup vs baseline: 1.0349x; 1.0349x over previous
import functools

import jax
import jax.numpy as jnp
from jax.experimental import pallas as pl
from jax.experimental.pallas import tpu as pltpu

N = 10000
K = 10
H = 128
DH = K * H
BLOCK_ROWS = 1000


def _cell_kernel(h_ref, c_ref, wf_ref, bf_ref, wiou_ref, biou_ref,
                 h_out_ref, c_out_ref):
    h_cat = h_ref[...].astype(jnp.bfloat16)
    u_res = jnp.dot(h_cat, wf_ref[...],
                    preferred_element_type=jnp.float32) + bf_ref[...]
    f = jax.nn.sigmoid(u_res)
    fc = f * c_ref[...]
    c_red = fc[:, 0:H]
    for k in range(1, K):
        c_red = c_red + fc[:, k * H:(k + 1) * H]
    iou = jnp.dot(h_cat, wiou_ref[...],
                  preferred_element_type=jnp.float32) + biou_ref[...]
    i = jax.nn.sigmoid(iou[:, 0:H])
    o = jax.nn.sigmoid(iou[:, H:2 * H])
    u = jnp.tanh(iou[:, 2 * H:3 * H])
    c_out = i * u + c_red
    c_out_ref[...] = c_out
    h_out_ref[...] = o * jnp.tanh(c_out)


@functools.partial(jax.jit, static_argnames=("interpret",))
def kernel(mailbox_h, mailbox_c, U_f_w, U_f_b, U_iou_w, b_iou,
           interpret=False):
    n = mailbox_h.shape[0]
    h2 = mailbox_h.reshape(n, DH)
    c2 = mailbox_c.reshape(n, DH)
    wf = U_f_w[:DH, :DH].astype(jnp.bfloat16)
    wiou_t = U_iou_w[:, :DH].T.astype(jnp.bfloat16)
    bf = U_f_b[:DH].reshape(1, DH)
    grid = (pl.cdiv(n, BLOCK_ROWS),)
    h_out, c_out = pl.pallas_call(
        _cell_kernel,
        grid=grid,
        in_specs=[
            pl.BlockSpec((BLOCK_ROWS, DH), lambda i: (i, 0)),
            pl.BlockSpec((BLOCK_ROWS, DH), lambda i: (i, 0)),
            pl.BlockSpec((DH, DH), lambda i: (0, 0)),
            pl.BlockSpec((1, DH), lambda i: (0, 0)),
            pl.BlockSpec((DH, 3 * H), lambda i: (0, 0)),
            pl.BlockSpec((1, 3 * H), lambda i: (0, 0)),
        ],
        out_specs=[
            pl.BlockSpec((BLOCK_ROWS, H), lambda i: (i, 0)),
            pl.BlockSpec((BLOCK_ROWS, H), lambda i: (i, 0)),
        ],
        out_shape=[
            jax.ShapeDtypeStruct((n, H), jnp.float32),
            jax.ShapeDtypeStruct((n, H), jnp.float32),
        ],
        compiler_params=pltpu.CompilerParams(
            dimension_semantics=("arbitrary",),
        ),
        interpret=interpret,
    )(h2, c2, wf, bf, wiou_t, b_iou)
    return (h_out, c_out)
